# baseline (speedup 1.0000x reference)
import functools

import jax
import jax.numpy as jnp
from jax import lax
from jax.experimental import pallas as pl
from jax.experimental.pallas import tpu as pltpu
from jax.experimental.pallas import tpu_sc as plsc

F32 = jnp.float32
I32 = jnp.int32

NC = 2
NS = 16
L = 16
NW = NC * NS
CH = 128
FRAC0 = 0.50
NGS = 4


def _split_chunks(total_per_pair):
  n0 = int(round(FRAC0 * total_per_pair / 2)) * 2
  n0 = max(4, min(total_per_pair - 4, n0))
  return n0, total_per_pair - n0


def _sc_conv_agg(nchunks, npad, n, d):
  assert d % L == 0 and npad % (NS * CH) == 0
  n0, n1 = _split_chunks(2 * nchunks)
  assert n0 % 2 == 0 and n1 % 2 == 0 and n0 >= 4 and n1 >= 4
  ndv = d // L
  rows_per_sub = npad // NS
  mesh = plsc.VectorSubcoreMesh(core_axis_name="c", subcore_axis_name="s")

  @functools.partial(
      pl.kernel,
      out_type=jax.ShapeDtypeStruct((NC, npad, d), F32),
      mesh=mesh,
      scratch_types=[
          pltpu.VMEM((2, CH), I32),
          pltpu.VMEM((2, CH), I32),
          pltpu.VMEM((2, CH), F32),
          pltpu.VMEM((2, CH, d), F32),
          pltpu.VMEM((d,), F32),
          pltpu.VMEM((d,), F32),
          pltpu.VMEM_SHARED((npad, d), F32),
          pltpu.SemaphoreType.DMA,
          pltpu.SemaphoreType.DMA,
          pltpu.SemaphoreType.DMA,
          pltpu.SemaphoreType.DMA,
          pltpu.SemaphoreType.DMA,
          pltpu.SemaphoreType.DMA,
      ],
  )
  def conv(table_h, src_h, dst_h, attr_h, w_h, b_h, out_h,
           src_v, dst_v, attr_v, rows, w_v, b_v, acc,
           g0, g1, sc0, sc1, i0, i1):
    c = lax.axis_index("c")
    s = lax.axis_index("s")
    gsem = (g0, g1)
    ssem = (sc0, sc1)
    isem = (i0, i1)

    pltpu.sync_copy(w_h, w_v)
    pltpu.sync_copy(b_h, b_v)
    abase = s * rows_per_sub

    def zrow(k, carry):
      z = jnp.zeros((L,), F32)
      for f in range(ndv):
        rows[0, k, pl.ds(f * L, L)] = z
      return carry
    lax.fori_loop(0, CH, zrow, 0)
    for i in range(rows_per_sub // CH):
      pltpu.sync_copy(rows.at[0], acc.at[pl.ds(abase + i * CH, CH)])
    plsc.subcore_barrier()

    wregs = [w_v[pl.ds(f * L, L)] for f in range(ndv)]
    bregs = [b_v[pl.ds(f * L, L)] for f in range(ndv)]

    def compute(buf):
      def group(g, carry):
        avec = attr_v[buf, pl.ds(g * L, L)]
        for i in range(L):
          a = jnp.full((L,), avec[i], F32)
          k = g * L + i
          for f in range(ndv):
            r = rows[buf, k, pl.ds(f * L, L)]
            rows[buf, k, pl.ds(f * L, L)] = jnp.maximum(
                r + a * wregs[f] + bregs[f], 0.0)
        return carry

      lax.fori_loop(0, CH // L, group, 0)

    def pipeline(nch, base):
      sl = pl.ds(0, CH)

      def start_idx(j, buf):
        pltpu.async_copy(src_h.at[base + j, sl], src_v.at[buf], isem[buf])
        pltpu.async_copy(dst_h.at[base + j, sl], dst_v.at[buf], isem[buf])
        pltpu.async_copy(attr_h.at[base + j, sl], attr_v.at[buf],
                         isem[buf])

      def wait_idx(buf):
        pltpu.make_async_copy(src_h.at[base, sl], src_v.at[buf],
                              isem[buf]).wait()
        pltpu.make_async_copy(dst_h.at[base, sl], dst_v.at[buf],
                              isem[buf]).wait()
        pltpu.make_async_copy(attr_h.at[base, sl], attr_v.at[buf],
                              isem[buf]).wait()

      sub = CH // NGS

      def start_gather(buf):
        for q in range(NGS):
          pltpu.async_copy(
              table_h.at[src_v.at[buf, pl.ds(q * sub, sub)]],
              rows.at[buf, pl.ds(q * sub, sub)], gsem[buf])

      def wait_gather(buf):
        for q in range(NGS):
          pltpu.make_async_copy(
              table_h.at[src_v.at[buf, pl.ds(q * sub, sub)]],
              rows.at[buf, pl.ds(q * sub, sub)], gsem[buf]).wait()

      def start_scatter(buf):
        pltpu.async_copy(rows.at[buf], acc.at[dst_v.at[buf]], ssem[buf],
                         add=True)

      def wait_scatter(buf):
        pltpu.make_async_copy(rows.at[buf], acc.at[dst_v.at[buf]],
                              ssem[buf]).wait()

      start_idx(0, 0)
      start_idx(1, 1)
      wait_idx(0)
      start_gather(0)

      def step(j2, carry):
        for buf in range(2):
          j = j2 * 2 + buf
          wait_gather(buf)
          compute(buf)
          start_scatter(buf)
          wait_idx(1 - buf)
          start_gather(1 - buf)
          wait_scatter(buf)
          start_idx(j + 2, buf)
        return carry

      lax.fori_loop(0, nch // 2 - 1, step, 0, unroll=False)
      wait_gather(0)
      compute(0)
      start_scatter(0)
      wait_idx(1)
      start_gather(1)
      wait_scatter(0)
      wait_gather(1)
      compute(1)
      start_scatter(1)
      wait_scatter(1)

    nch = jnp.where(c == 0, n0, n1)
    base = jnp.where(c == 0, s * n0, NS * n0 + s * n1)
    pipeline(nch, base)

    plsc.subcore_barrier()
    pltpu.sync_copy(acc.at[pl.ds(abase, rows_per_sub)],
                    out_h.at[c, pl.ds(abase, rows_per_sub), pl.ds(0, d)])

  return conv


def _bn(h, g, be):
  mu = jnp.mean(h, axis=0)
  var = jnp.mean((h - mu) ** 2, axis=0)
  return g * (h - mu) / jnp.sqrt(var + 1e-5) + be


def _rowmat(x, W):
  return lax.dot_general(x, W, (((1,), (1,)), ((), ())),
                         preferred_element_type=F32)


def _tc_conv_mlp(x, agg, W, b, g, be):
  n = x.shape[0]
  dout = W.shape[0]

  def body(x_ref, agg_ref, w_ref, b_ref, g_ref, be_ref, o_ref):
    out = x_ref[...] + agg_ref[0, :n, :] + agg_ref[1, :n, :]
    h = jnp.maximum(_rowmat(out, w_ref[...]) + b_ref[...][None, :], 0.0)
    o_ref[...] = _bn(h, g_ref[...][None, :], be_ref[...][None, :])

  return pl.pallas_call(
      body, out_shape=jax.ShapeDtypeStruct((n, dout), F32))(
          x, agg, W, b, g, be)


def _tc_tail(x1, agg2, W2, b2, g2, be2, W3a, W3b, b3, g3, be3,
             W4, b4, g4, be4, W5, b5, g5, be5):
  n = x1.shape[0]
  cdim = W5.shape[0]

  def body(x1_ref, agg_ref, w2_ref, b2_ref, g2_ref, be2_ref,
           w3a_ref, w3b_ref, b3_ref, g3_ref, be3_ref,
           w4_ref, b4_ref, g4_ref, be4_ref,
           w5_ref, b5_ref, g5_ref, be5_ref, o_ref):
    x1v = x1_ref[...]
    out2 = x1v + agg_ref[0, :n, :] + agg_ref[1, :n, :]
    h2 = jnp.maximum(_rowmat(out2, w2_ref[...]) + b2_ref[...][None, :], 0.0)
    x2 = _bn(h2, g2_ref[...][None, :], be2_ref[...][None, :])
    h3 = jnp.maximum(
        _rowmat(x1v, w3a_ref[...]) + _rowmat(x2, w3b_ref[...])
        + b3_ref[...][None, :], 0.0)
    x3 = _bn(h3, g3_ref[...][None, :], be3_ref[...][None, :])
    h4 = jnp.maximum(_rowmat(x3, w4_ref[...]) + b4_ref[...][None, :], 0.0)
    x4 = _bn(h4, g4_ref[...][None, :], be4_ref[...][None, :])
    h5 = jnp.maximum(_rowmat(x4, w5_ref[...]) + b5_ref[...][None, :], 0.0)
    o_ref[...] = _bn(h5, g5_ref[...][None, :], be5_ref[...][None, :])

  return pl.pallas_call(
      body, out_shape=jax.ShapeDtypeStruct((n, cdim), F32))(
          x1, agg2, W2, b2, g2, be2, W3a, W3b, b3, g3, be3,
          W4, b4, g4, be4, W5, b5, g5, be5)


def kernel(x, edge_index, edge_attr, W_lin, b_lin, W1, b1, g1, be1,
           W2, b2, g2, be2, W3, b3, g3, be3, W4, b4, g4, be4,
           W5, b5, g5, be5):
  n, d = x.shape
  e = edge_index.shape[1]

  per = NW * CH
  nchunks = -(-e // per)
  nchunks += nchunks % 2
  nchunks = max(nchunks, 8)
  epad = nchunks * per
  tchunks = epad // CH
  npad = -(-n // (NS * CH)) * (NS * CH)

  pad = epad - e
  srcF = jnp.concatenate(
      [edge_index[0], jnp.zeros((pad,), I32)]).reshape(tchunks, CH)
  dstF = jnp.concatenate(
      [edge_index[1], jnp.full((pad,), n, I32)]).reshape(tchunks, CH)
  attrF = jnp.concatenate(
      [edge_attr[:, 0], jnp.zeros((pad,), F32)]).reshape(tchunks, CH)
  w_col = W_lin[:, 0]

  conv = _sc_conv_agg(nchunks, npad, n, d)
  agg1 = conv(x, srcF, dstF, attrF, w_col, b_lin)
  x1 = _tc_conv_mlp(x, agg1, W1, b1, g1, be1)
  agg2 = conv(x1, srcF, dstF, attrF, w_col, b_lin)
  return _tc_tail(x1, agg2, W2, b2, g2, be2,
                  W3[:, :d], W3[:, d:], b3, g3, be3,
                  W4, b4, g4, be4, W5, b5, g5, be5)


# speedup vs baseline: 1.1327x; 1.1327x over previous
import functools

import jax
import jax.numpy as jnp
from jax import lax
from jax.experimental import pallas as pl
from jax.experimental.pallas import tpu as pltpu
from jax.experimental.pallas import tpu_sc as plsc

F32 = jnp.float32
I32 = jnp.int32

NC = 2
NS = 16
L = 16
NW = NC * NS
CH = 128
FRAC0 = 0.70


def _split_chunks(total_per_pair):
  n0 = int(round(FRAC0 * total_per_pair / 2)) * 2
  n0 = max(4, min(total_per_pair - 4, n0))
  return n0, total_per_pair - n0


def _sc_conv_agg(nchunks, npad, n, d):
  assert d % L == 0 and npad % (NS * CH) == 0
  n0, n1 = _split_chunks(2 * nchunks)
  assert n0 % 2 == 0 and n1 % 2 == 0 and n0 >= 4 and n1 >= 4
  ndv = d // L
  rows_per_sub = npad // NS
  mesh = plsc.VectorSubcoreMesh(core_axis_name="c", subcore_axis_name="s")

  @functools.partial(
      pl.kernel,
      out_type=jax.ShapeDtypeStruct((NC, npad, d), F32),
      mesh=mesh,
      scratch_types=[
          pltpu.VMEM((2, CH), I32),
          pltpu.VMEM((2, CH), I32),
          pltpu.VMEM((2, CH), F32),
          pltpu.VMEM((2, CH, d), F32),
          pltpu.VMEM((d,), F32),
          pltpu.VMEM((d,), F32),
          pltpu.VMEM_SHARED((npad, d), F32),
          pltpu.SemaphoreType.DMA,
          pltpu.SemaphoreType.DMA,
          pltpu.SemaphoreType.DMA,
          pltpu.SemaphoreType.DMA,
          pltpu.SemaphoreType.DMA,
          pltpu.SemaphoreType.DMA,
      ],
  )
  def conv(table_h, src_h, dst_h, attr_h, w_h, b_h, out_h,
           src_v, dst_v, attr_v, rows, w_v, b_v, acc,
           g0, g1, sc0, sc1, i0, i1):
    c = lax.axis_index("c")
    s = lax.axis_index("s")
    gsem = (g0, g1)
    ssem = (sc0, sc1)
    isem = (i0, i1)

    pltpu.sync_copy(w_h, w_v)
    pltpu.sync_copy(b_h, b_v)
    abase = s * rows_per_sub

    def zrow(k, carry):
      z = jnp.zeros((L,), F32)
      for f in range(ndv):
        rows[0, k, pl.ds(f * L, L)] = z
      return carry
    lax.fori_loop(0, CH, zrow, 0)
    for i in range(rows_per_sub // CH):
      pltpu.sync_copy(rows.at[0], acc.at[pl.ds(abase + i * CH, CH)])
    plsc.subcore_barrier()

    wregs = [w_v[pl.ds(f * L, L)] for f in range(ndv)]
    bregs = [b_v[pl.ds(f * L, L)] for f in range(ndv)]

    def compute(buf):
      def group(g, carry):
        avec = attr_v[buf, pl.ds(g * L, L)]
        for i in range(L):
          a = jnp.full((L,), avec[i], F32)
          k = g * L + i
          for f in range(ndv):
            r = rows[buf, k, pl.ds(f * L, L)]
            rows[buf, k, pl.ds(f * L, L)] = jnp.maximum(
                r + a * wregs[f] + bregs[f], 0.0)
        return carry

      lax.fori_loop(0, CH // L, group, 0)

    def pipeline(nch, base):
      sl = pl.ds(0, CH)

      def start_idx(j, buf):
        pltpu.async_copy(src_h.at[base + j, sl], src_v.at[buf], isem[buf])
        pltpu.async_copy(dst_h.at[base + j, sl], dst_v.at[buf], isem[buf])
        pltpu.async_copy(attr_h.at[base + j, sl], attr_v.at[buf],
                         isem[buf])

      def wait_idx(buf):
        pltpu.make_async_copy(src_h.at[base, sl], src_v.at[buf],
                              isem[buf]).wait()
        pltpu.make_async_copy(dst_h.at[base, sl], dst_v.at[buf],
                              isem[buf]).wait()
        pltpu.make_async_copy(attr_h.at[base, sl], attr_v.at[buf],
                              isem[buf]).wait()

      def start_gather(buf):
        pltpu.async_copy(table_h.at[src_v.at[buf]], rows.at[buf],
                         gsem[buf])

      def wait_gather(buf):
        pltpu.make_async_copy(table_h.at[src_v.at[buf]], rows.at[buf],
                              gsem[buf]).wait()

      def start_scatter(buf):
        pltpu.async_copy(rows.at[buf], acc.at[dst_v.at[buf]], ssem[buf],
                         add=True)

      def wait_scatter(buf):
        pltpu.make_async_copy(rows.at[buf], acc.at[dst_v.at[buf]],
                              ssem[buf]).wait()

      start_idx(0, 0)
      start_idx(1, 1)
      wait_idx(0)
      start_gather(0)

      def step(j2, carry):
        for buf in range(2):
          j = j2 * 2 + buf
          wait_gather(buf)
          compute(buf)
          start_scatter(buf)
          wait_idx(1 - buf)
          start_gather(1 - buf)
          wait_scatter(buf)
          start_idx(j + 2, buf)
        return carry

      lax.fori_loop(0, nch // 2, step, 0, unroll=False)
      wait_gather(0)
      wait_idx(1)

    nch = jnp.where(c == 0, n0, n1)
    base = jnp.where(c == 0, s * n0, NS * n0 + s * n1)
    pipeline(nch, base)

    plsc.subcore_barrier()
    pltpu.sync_copy(acc.at[pl.ds(abase, rows_per_sub)],
                    out_h.at[c, pl.ds(abase, rows_per_sub), pl.ds(0, d)])

  return conv


def _bn(h, g, be):
  mu = jnp.mean(h, axis=0)
  var = jnp.mean((h - mu) ** 2, axis=0)
  return g * (h - mu) / jnp.sqrt(var + 1e-5) + be


def _rowmat(x, W):
  return lax.dot_general(x, W, (((1,), (1,)), ((), ())),
                         preferred_element_type=F32)


def _tc_conv_mlp(x, agg, W, b, g, be):
  n = x.shape[0]
  dout = W.shape[0]

  def body(x_ref, agg_ref, w_ref, b_ref, g_ref, be_ref, o_ref):
    out = x_ref[...] + agg_ref[0, :n, :] + agg_ref[1, :n, :]
    h = jnp.maximum(_rowmat(out, w_ref[...]) + b_ref[...][None, :], 0.0)
    o_ref[...] = _bn(h, g_ref[...][None, :], be_ref[...][None, :])

  return pl.pallas_call(
      body, out_shape=jax.ShapeDtypeStruct((n, dout), F32))(
          x, agg, W, b, g, be)


def _tc_tail(x1, agg2, W2, b2, g2, be2, W3a, W3b, b3, g3, be3,
             W4, b4, g4, be4, W5, b5, g5, be5):
  n = x1.shape[0]
  cdim = W5.shape[0]

  def body(x1_ref, agg_ref, w2_ref, b2_ref, g2_ref, be2_ref,
           w3a_ref, w3b_ref, b3_ref, g3_ref, be3_ref,
           w4_ref, b4_ref, g4_ref, be4_ref,
           w5_ref, b5_ref, g5_ref, be5_ref, o_ref):
    x1v = x1_ref[...]
    out2 = x1v + agg_ref[0, :n, :] + agg_ref[1, :n, :]
    h2 = jnp.maximum(_rowmat(out2, w2_ref[...]) + b2_ref[...][None, :], 0.0)
    x2 = _bn(h2, g2_ref[...][None, :], be2_ref[...][None, :])
    h3 = jnp.maximum(
        _rowmat(x1v, w3a_ref[...]) + _rowmat(x2, w3b_ref[...])
        + b3_ref[...][None, :], 0.0)
    x3 = _bn(h3, g3_ref[...][None, :], be3_ref[...][None, :])
    h4 = jnp.maximum(_rowmat(x3, w4_ref[...]) + b4_ref[...][None, :], 0.0)
    x4 = _bn(h4, g4_ref[...][None, :], be4_ref[...][None, :])
    h5 = jnp.maximum(_rowmat(x4, w5_ref[...]) + b5_ref[...][None, :], 0.0)
    o_ref[...] = _bn(h5, g5_ref[...][None, :], be5_ref[...][None, :])

  return pl.pallas_call(
      body, out_shape=jax.ShapeDtypeStruct((n, cdim), F32))(
          x1, agg2, W2, b2, g2, be2, W3a, W3b, b3, g3, be3,
          W4, b4, g4, be4, W5, b5, g5, be5)


def kernel(x, edge_index, edge_attr, W_lin, b_lin, W1, b1, g1, be1,
           W2, b2, g2, be2, W3, b3, g3, be3, W4, b4, g4, be4,
           W5, b5, g5, be5):
  n, d = x.shape
  e = edge_index.shape[1]

  per = NW * CH
  nchunks = -(-e // per)
  nchunks += nchunks % 2
  nchunks = max(nchunks, 8)
  epad = nchunks * per
  tchunks = epad // CH
  npad = -(-n // (NS * CH)) * (NS * CH)

  pad = epad - e + 2 * CH
  srcF = jnp.concatenate(
      [edge_index[0], jnp.zeros((pad,), I32)]).reshape(tchunks + 2, CH)
  dstF = jnp.concatenate(
      [edge_index[1], jnp.full((pad,), n, I32)]).reshape(tchunks + 2, CH)
  attrF = jnp.concatenate(
      [edge_attr[:, 0], jnp.zeros((pad,), F32)]).reshape(tchunks + 2, CH)
  w_col = W_lin[:, 0]

  conv = _sc_conv_agg(nchunks, npad, n, d)
  agg1 = conv(x, srcF, dstF, attrF, w_col, b_lin)
  x1 = _tc_conv_mlp(x, agg1, W1, b1, g1, be1)
  agg2 = conv(x1, srcF, dstF, attrF, w_col, b_lin)
  return _tc_tail(x1, agg2, W2, b2, g2, be2,
                  W3[:, :d], W3[:, d:], b3, g3, be3,
                  W4, b4, g4, be4, W5, b5, g5, be5)


# speedup vs baseline: 1.2132x; 1.0710x over previous
import functools

import jax
import jax.numpy as jnp
from jax import lax
from jax.experimental import pallas as pl
from jax.experimental.pallas import tpu as pltpu
from jax.experimental.pallas import tpu_sc as plsc

F32 = jnp.float32
I32 = jnp.int32

NC = 2
NS = 16
L = 16
NW = NC * NS
CH = 128
FRAC0 = 0.79


def _split_chunks(total_per_pair):
  n0 = int(round(FRAC0 * total_per_pair / 2)) * 2
  n0 = max(4, min(total_per_pair - 4, n0))
  return n0, total_per_pair - n0


def _sc_conv_agg(nchunks, npad, n, d):
  assert d % L == 0 and npad % (NS * CH) == 0
  n0, n1 = _split_chunks(2 * nchunks)
  assert n0 % 2 == 0 and n1 % 2 == 0 and n0 >= 4 and n1 >= 4
  ndv = d // L
  rows_per_sub = npad // NS
  mesh = plsc.VectorSubcoreMesh(core_axis_name="c", subcore_axis_name="s")

  @functools.partial(
      pl.kernel,
      out_type=jax.ShapeDtypeStruct((NC, npad, d), F32),
      mesh=mesh,
      scratch_types=[
          pltpu.VMEM((2, CH), I32),
          pltpu.VMEM((2, CH), I32),
          pltpu.VMEM((2, CH), F32),
          pltpu.VMEM((2, CH, d), F32),
          pltpu.VMEM((d,), F32),
          pltpu.VMEM((d,), F32),
          pltpu.VMEM_SHARED((npad, d), F32),
          pltpu.SemaphoreType.DMA,
          pltpu.SemaphoreType.DMA,
          pltpu.SemaphoreType.DMA,
          pltpu.SemaphoreType.DMA,
          pltpu.SemaphoreType.DMA,
          pltpu.SemaphoreType.DMA,
      ],
  )
  def conv(table_h, src_h, dst_h, attr_h, w_h, b_h, out_h,
           src_v, dst_v, attr_v, rows, w_v, b_v, acc,
           g0, g1, sc0, sc1, i0, i1):
    c = lax.axis_index("c")
    s = lax.axis_index("s")
    gsem = (g0, g1)
    ssem = (sc0, sc1)
    isem = (i0, i1)

    pltpu.sync_copy(w_h, w_v)
    pltpu.sync_copy(b_h, b_v)
    abase = s * rows_per_sub

    def zrow(k, carry):
      z = jnp.zeros((L,), F32)
      for f in range(ndv):
        rows[0, k, pl.ds(f * L, L)] = z
      return carry
    lax.fori_loop(0, CH, zrow, 0)
    for i in range(rows_per_sub // CH):
      pltpu.sync_copy(rows.at[0], acc.at[pl.ds(abase + i * CH, CH)])
    plsc.subcore_barrier()

    wregs = [w_v[pl.ds(f * L, L)] for f in range(ndv)]
    bregs = [b_v[pl.ds(f * L, L)] for f in range(ndv)]

    def compute(buf):
      def group(g, carry):
        avec = attr_v[buf, pl.ds(g * L, L)]
        for i in range(L):
          a = jnp.full((L,), avec[i], F32)
          k = g * L + i
          for f in range(ndv):
            r = rows[buf, k, pl.ds(f * L, L)]
            rows[buf, k, pl.ds(f * L, L)] = jnp.maximum(
                r + a * wregs[f] + bregs[f], 0.0)
        return carry

      lax.fori_loop(0, CH // L, group, 0)

    def pipeline(nch, base):
      sl = pl.ds(0, CH)

      def start_idx(j, buf):
        pltpu.async_copy(src_h.at[base + j, sl], src_v.at[buf], isem[buf])
        pltpu.async_copy(dst_h.at[base + j, sl], dst_v.at[buf], isem[buf])
        pltpu.async_copy(attr_h.at[base + j, sl], attr_v.at[buf],
                         isem[buf])

      def wait_idx(buf):
        pltpu.make_async_copy(src_h.at[base, sl], src_v.at[buf],
                              isem[buf]).wait()
        pltpu.make_async_copy(dst_h.at[base, sl], dst_v.at[buf],
                              isem[buf]).wait()
        pltpu.make_async_copy(attr_h.at[base, sl], attr_v.at[buf],
                              isem[buf]).wait()

      def start_gather(buf):
        pltpu.async_copy(table_h.at[src_v.at[buf]], rows.at[buf],
                         gsem[buf])

      def wait_gather(buf):
        pltpu.make_async_copy(table_h.at[src_v.at[buf]], rows.at[buf],
                              gsem[buf]).wait()

      def start_scatter(buf):
        pltpu.async_copy(rows.at[buf], acc.at[dst_v.at[buf]], ssem[buf],
                         add=True)

      def wait_scatter(buf):
        pltpu.make_async_copy(rows.at[buf], acc.at[dst_v.at[buf]],
                              ssem[buf]).wait()

      start_idx(0, 0)
      start_idx(1, 1)
      wait_idx(0)
      start_gather(0)

      def step(j2, carry):
        for buf in range(2):
          j = j2 * 2 + buf
          wait_gather(buf)
          compute(buf)
          start_scatter(buf)
          wait_idx(1 - buf)
          start_gather(1 - buf)
          wait_scatter(buf)
          start_idx(j + 2, buf)
        return carry

      lax.fori_loop(0, nch // 2, step, 0, unroll=False)
      wait_gather(0)
      wait_idx(1)

    nch = jnp.where(c == 0, n0, n1)
    base = jnp.where(c == 0, s * n0, NS * n0 + s * n1)
    pipeline(nch, base)

    plsc.subcore_barrier()
    pltpu.sync_copy(acc.at[pl.ds(abase, rows_per_sub)],
                    out_h.at[c, pl.ds(abase, rows_per_sub), pl.ds(0, d)])

  return conv


def _bn(h, g, be):
  mu = jnp.mean(h, axis=0)
  var = jnp.mean((h - mu) ** 2, axis=0)
  return g * (h - mu) / jnp.sqrt(var + 1e-5) + be


def _rowmat(x, W):
  return lax.dot_general(x, W, (((1,), (1,)), ((), ())),
                         preferred_element_type=F32)


def _tc_conv_mlp(x, agg, W, b, g, be):
  n = x.shape[0]
  dout = W.shape[0]

  def body(x_ref, agg_ref, w_ref, b_ref, g_ref, be_ref, o_ref):
    out = x_ref[...] + agg_ref[0, :n, :] + agg_ref[1, :n, :]
    h = jnp.maximum(_rowmat(out, w_ref[...]) + b_ref[...][None, :], 0.0)
    o_ref[...] = _bn(h, g_ref[...][None, :], be_ref[...][None, :])

  return pl.pallas_call(
      body, out_shape=jax.ShapeDtypeStruct((n, dout), F32))(
          x, agg, W, b, g, be)


def _tc_tail(x1, agg2, W2, b2, g2, be2, W3a, W3b, b3, g3, be3,
             W4, b4, g4, be4, W5, b5, g5, be5):
  n = x1.shape[0]
  cdim = W5.shape[0]

  def body(x1_ref, agg_ref, w2_ref, b2_ref, g2_ref, be2_ref,
           w3a_ref, w3b_ref, b3_ref, g3_ref, be3_ref,
           w4_ref, b4_ref, g4_ref, be4_ref,
           w5_ref, b5_ref, g5_ref, be5_ref, o_ref):
    x1v = x1_ref[...]
    out2 = x1v + agg_ref[0, :n, :] + agg_ref[1, :n, :]
    h2 = jnp.maximum(_rowmat(out2, w2_ref[...]) + b2_ref[...][None, :], 0.0)
    x2 = _bn(h2, g2_ref[...][None, :], be2_ref[...][None, :])
    h3 = jnp.maximum(
        _rowmat(x1v, w3a_ref[...]) + _rowmat(x2, w3b_ref[...])
        + b3_ref[...][None, :], 0.0)
    x3 = _bn(h3, g3_ref[...][None, :], be3_ref[...][None, :])
    h4 = jnp.maximum(_rowmat(x3, w4_ref[...]) + b4_ref[...][None, :], 0.0)
    x4 = _bn(h4, g4_ref[...][None, :], be4_ref[...][None, :])
    h5 = jnp.maximum(_rowmat(x4, w5_ref[...]) + b5_ref[...][None, :], 0.0)
    o_ref[...] = _bn(h5, g5_ref[...][None, :], be5_ref[...][None, :])

  return pl.pallas_call(
      body, out_shape=jax.ShapeDtypeStruct((n, cdim), F32))(
          x1, agg2, W2, b2, g2, be2, W3a, W3b, b3, g3, be3,
          W4, b4, g4, be4, W5, b5, g5, be5)


def kernel(x, edge_index, edge_attr, W_lin, b_lin, W1, b1, g1, be1,
           W2, b2, g2, be2, W3, b3, g3, be3, W4, b4, g4, be4,
           W5, b5, g5, be5):
  n, d = x.shape
  e = edge_index.shape[1]

  per = NW * CH
  nchunks = -(-e // per)
  nchunks += nchunks % 2
  nchunks = max(nchunks, 8)
  epad = nchunks * per
  tchunks = epad // CH
  npad = -(-n // (NS * CH)) * (NS * CH)

  pad = epad - e + 2 * CH
  srcF = jnp.concatenate(
      [edge_index[0], jnp.zeros((pad,), I32)]).reshape(tchunks + 2, CH)
  dstF = jnp.concatenate(
      [edge_index[1], jnp.full((pad,), n, I32)]).reshape(tchunks + 2, CH)
  attrF = jnp.concatenate(
      [edge_attr[:, 0], jnp.zeros((pad,), F32)]).reshape(tchunks + 2, CH)
  w_col = W_lin[:, 0]

  conv = _sc_conv_agg(nchunks, npad, n, d)
  agg1 = conv(x, srcF, dstF, attrF, w_col, b_lin)
  x1 = _tc_conv_mlp(x, agg1, W1, b1, g1, be1)
  agg2 = conv(x1, srcF, dstF, attrF, w_col, b_lin)
  return _tc_tail(x1, agg2, W2, b2, g2, be2,
                  W3[:, :d], W3[:, d:], b3, g3, be3,
                  W4, b4, g4, be4, W5, b5, g5, be5)
